# baseline (speedup 1.0000x reference)
import functools

import jax
import jax.numpy as jnp
from jax import lax
from jax.experimental import pallas as pl
from jax.experimental.pallas import tpu as pltpu
from jax.experimental.pallas import tpu_sc as plsc

MAX_CONTEXT_LENGTH = 8192
D_MODEL = 1024
BATCH = 4
SEQ_LEN = 8192

NTOT = BATCH * SEQ_LEN
N_TC = 32768
N_SC = NTOT - N_TC


NW = 32
CHUNK = 16
NBUF = 4

if N_SC:
    B_PER_W = N_SC // NW
    NCHUNK = B_PER_W // CHUNK
    NROUND = NCHUNK // NBUF

    _mesh = plsc.VectorSubcoreMesh(core_axis_name="c", subcore_axis_name="s")

    @functools.partial(
        pl.kernel,
        mesh=_mesh,
        out_type=jax.ShapeDtypeStruct((N_SC, D_MODEL), jnp.float32),
        scratch_types=[
            pltpu.VMEM((B_PER_W,), jnp.int32),
            pltpu.VMEM((NBUF, CHUNK, D_MODEL), jnp.float32),
        ]
        + [pltpu.SemaphoreType.DMA] * (2 * NBUF),
    )
    def _sc_gather(ids_hbm, table_hbm, out_hbm, idx_v, rows_v, *sems):
        gsem, osem = sems[:NBUF], sems[NBUF:]
        wid = lax.axis_index("s") * 2 + lax.axis_index("c")
        base = wid * B_PER_W
        pltpu.sync_copy(ids_hbm.at[pl.ds(base, B_PER_W)], idx_v)

        def gather(g, buf):
            return pltpu.make_async_copy(
                table_hbm.at[idx_v.at[pl.ds(g * CHUNK, CHUNK)]],
                rows_v.at[buf],
                gsem[buf],
            )

        def writeback(g, buf):
            return pltpu.make_async_copy(
                rows_v.at[buf],
                out_hbm.at[pl.ds(base + g * CHUNK, CHUNK)],
                osem[buf],
            )

        gather(0, 0).start()
        gather(1, 1).start()

        def step(g, j, wait_wb, start_g):
            gather(g, j).wait()
            if wait_wb:
                writeback(g - 2, (j + 2) % NBUF).wait()
            if start_g:
                gather(g + 2, (j + 2) % NBUF).start()
            writeback(g, j).start()

        for j in range(NBUF):
            step(j, j, j >= 2, True)

        def round_body(r, c):
            g0 = NBUF * r
            for j in range(NBUF):
                step(g0 + j, j, True, True)
            return c

        lax.fori_loop(1, NROUND - 1, round_body, 0)

        g0 = NBUF * (NROUND - 1)
        for j in range(NBUF):
            step(g0 + j, j, True, j < 2)
        writeback(NCHUNK - 2, 2).wait()
        writeback(NCHUNK - 1, 3).wait()


RG = 64

if N_TC:

    def _tc_body(idx_ref, table_ref, out_ref):
        i = pl.program_id(0)

        def body(j, c):
            out_ref[j] = table_ref[idx_ref[i * RG + j]]
            return c

        lax.fori_loop(0, RG, body, 0, unroll=8)

    _tc_gather = pl.pallas_call(
        _tc_body,
        grid_spec=pltpu.PrefetchScalarGridSpec(
            num_scalar_prefetch=1,
            grid=(N_TC // RG,),
            in_specs=[
                pl.BlockSpec(
                    (MAX_CONTEXT_LENGTH, 8, 128), lambda i, idx: (0, 0, 0)
                )
            ],
            out_specs=pl.BlockSpec((RG, 8, 128), lambda i, idx: (i, 0, 0)),
        ),
        out_shape=jax.ShapeDtypeStruct((N_TC, 8, 128), jnp.float32),
    )


def kernel(position_ids, table):
    ids_flat = position_ids.reshape(-1).astype(jnp.int32)
    parts = []
    if N_TC:
        out_tc = _tc_gather(
            ids_flat[:N_TC], table.reshape(MAX_CONTEXT_LENGTH, 8, 128)
        )
        parts.append(out_tc.reshape(N_TC, D_MODEL))
    if N_SC:
        parts.append(_sc_gather(ids_flat[N_TC:], table))
    out = parts[0] if len(parts) == 1 else jnp.concatenate(parts, axis=0)
    return out.reshape(BATCH, SEQ_LEN, D_MODEL)


# speedup vs baseline: 3.4231x; 3.4231x over previous
import functools

import jax
import jax.numpy as jnp
from jax import lax
from jax.experimental import pallas as pl
from jax.experimental.pallas import tpu as pltpu
from jax.experimental.pallas import tpu_sc as plsc

MAX_CONTEXT_LENGTH = 8192
D_MODEL = 1024
BATCH = 4
SEQ_LEN = 8192

NTOT = BATCH * SEQ_LEN
NW = 32
B_PER_W = NTOT // NW
CHUNK = 32
NBUF = 3
NCHUNK = B_PER_W // CHUNK

_mesh = plsc.VectorSubcoreMesh(core_axis_name="c", subcore_axis_name="s")


@functools.partial(
    pl.kernel,
    mesh=_mesh,
    out_type=jax.ShapeDtypeStruct((NTOT, D_MODEL), jnp.float32),
    scratch_types=[
        pltpu.VMEM((B_PER_W,), jnp.int32),
        pltpu.VMEM((NBUF, CHUNK, D_MODEL), jnp.float32),
    ]
    + [pltpu.SemaphoreType.DMA] * (2 * NBUF),
)
def _sc_gather(ids_hbm, table_hbm, out_hbm, idx_v, rows_v, *sems):
    gsem, osem = sems[:NBUF], sems[NBUF:]
    wid = lax.axis_index("s") * 2 + lax.axis_index("c")
    base = wid * B_PER_W
    pltpu.sync_copy(ids_hbm.at[pl.ds(base, B_PER_W)], idx_v)

    def gather(g, buf):
        return pltpu.make_async_copy(
            table_hbm.at[idx_v.at[pl.ds(g * CHUNK, CHUNK)]],
            rows_v.at[buf],
            gsem[buf],
        )

    def writeback(g, buf):
        return pltpu.make_async_copy(
            rows_v.at[buf],
            out_hbm.at[pl.ds(base + g * CHUNK, CHUNK)],
            osem[buf],
        )

    gather(0, 0).start()

    def step(g, j, wait_wb, start_g):
        gather(g, j).wait()
        if wait_wb:
            writeback(g - 2, (j + 1) % NBUF).wait()
        if start_g:
            gather(g + 1, (j + 1) % NBUF).start()
        writeback(g, j).start()

    for j in range(NBUF):
        step(j, j, j >= 2, True)

    def round_body(r, c):
        g0 = NBUF * r
        for j in range(NBUF):
            step(g0 + j, j, True, True)
        return c

    nround = NCHUNK // NBUF
    rem = NCHUNK - nround * NBUF
    lax.fori_loop(1, nround, round_body, 0)

    g0 = NBUF * nround
    for j in range(rem):
        step(g0 + j, j, True, j < rem - 1)
    for g in range(NCHUNK - 2, NCHUNK):
        writeback(g, g % NBUF).wait()


def kernel(position_ids, table):
    ids_flat = position_ids.reshape(-1).astype(jnp.int32)
    out = _sc_gather(ids_flat, table)
    return out.reshape(BATCH, SEQ_LEN, D_MODEL)


# speedup vs baseline: 3.5738x; 1.0440x over previous
import functools

import jax
import jax.numpy as jnp
from jax import lax
from jax.experimental import pallas as pl
from jax.experimental.pallas import tpu as pltpu
from jax.experimental.pallas import tpu_sc as plsc

MAX_CONTEXT_LENGTH = 8192
D_MODEL = 1024
BATCH = 4
SEQ_LEN = 8192

NTOT = BATCH * SEQ_LEN
NW = 32
B_PER_W = NTOT // NW
CHUNK = 16
NBUF = 6
LOOKAHEAD = 3
NCHUNK = B_PER_W // CHUNK

_mesh = plsc.VectorSubcoreMesh(core_axis_name="c", subcore_axis_name="s")


@functools.partial(
    pl.kernel,
    mesh=_mesh,
    out_type=jax.ShapeDtypeStruct((NTOT, D_MODEL), jnp.float32),
    scratch_types=[
        pltpu.VMEM((B_PER_W,), jnp.int32),
        pltpu.VMEM((NBUF, CHUNK, D_MODEL), jnp.float32),
    ]
    + [pltpu.SemaphoreType.DMA] * (2 * NBUF),
)
def _sc_gather(ids_hbm, table_hbm, out_hbm, idx_v, rows_v, *sems):
    gsem, osem = sems[:NBUF], sems[NBUF:]
    wid = lax.axis_index("s") * 2 + lax.axis_index("c")
    base = wid * B_PER_W
    pltpu.sync_copy(ids_hbm.at[pl.ds(base, B_PER_W)], idx_v)

    def gather(g, buf):
        return pltpu.make_async_copy(
            table_hbm.at[idx_v.at[pl.ds(g * CHUNK, CHUNK)]],
            rows_v.at[buf],
            gsem[buf],
        )

    def writeback(g, buf):
        return pltpu.make_async_copy(
            rows_v.at[buf],
            out_hbm.at[pl.ds(base + g * CHUNK, CHUNK)],
            osem[buf],
        )

    for g in range(LOOKAHEAD):
        gather(g, g).start()

    def step(g, j, wait_wb, start_g):
        gather(g, j).wait()
        if start_g:
            if wait_wb:
                writeback(g + LOOKAHEAD - NBUF, (j + LOOKAHEAD) % NBUF).wait()
            gather(g + LOOKAHEAD, (j + LOOKAHEAD) % NBUF).start()
        writeback(g, j).start()

    for j in range(NBUF):
        step(j, j, j >= NBUF - LOOKAHEAD, True)

    def round_body(r, c):
        g0 = NBUF * r
        for j in range(NBUF):
            step(g0 + j, j, True, True)
        return c

    nround = NCHUNK // NBUF
    lax.fori_loop(1, nround, round_body, 0)

    g0 = NBUF * nround
    for j in range(NCHUNK - g0):
        g = g0 + j
        step(g, j, True, g + LOOKAHEAD < NCHUNK)
    for g in range(NCHUNK - NBUF, NCHUNK):
        writeback(g, g % NBUF).wait()


def kernel(position_ids, table):
    ids_flat = position_ids.reshape(-1).astype(jnp.int32)
    out = _sc_gather(ids_flat, table)
    return out.reshape(BATCH, SEQ_LEN, D_MODEL)
